# baseline (speedup 1.0000x reference)
import functools

import jax
import jax.numpy as jnp
from jax import lax
from jax.experimental import pallas as pl
from jax.experimental.pallas import tpu as pltpu
from jax.experimental.pallas import tpu_sc as plsc

_TIME_STEPS = 1000
_DIM = 512
_BATCH = 4096


@functools.cache
def _build_gather():
    info = plsc.get_sparse_core_info()
    num_cores, num_subcores = info.num_cores, info.num_subcores
    num_workers = num_cores * num_subcores
    rows_per_worker = _BATCH // num_workers
    assert _BATCH % (8 * num_workers) == 0

    mesh = plsc.VectorSubcoreMesh(core_axis_name="c", subcore_axis_name="s")

    chunk = 16
    n_chunks = rows_per_worker // chunk

    @functools.partial(
        pl.kernel,
        mesh=mesh,
        out_type=jax.ShapeDtypeStruct((_BATCH, _DIM // 128, 1, 128),
                                      jnp.float32),
        scratch_types=[
            pltpu.VMEM((rows_per_worker,), jnp.int32),
            pltpu.VMEM((rows_per_worker, _DIM // 128, 1, 128), jnp.float32),
            pltpu.SemaphoreType.DMA,
            [pltpu.SemaphoreType.DMA] * n_chunks,
            [pltpu.SemaphoreType.DMA] * n_chunks,
        ],
    )
    def gather_kernel(idx_hbm, table_hbm, out_hbm, idx_v, rows_v, isem,
                      gsems, ssems):
        wid = lax.axis_index("s") * num_cores + lax.axis_index("c")
        base = wid * rows_per_worker
        pltpu.async_copy(idx_hbm.at[pl.ds(base, rows_per_worker)], idx_v,
                         isem).wait()
        gathers = [
            pltpu.async_copy(table_hbm.at[idx_v.at[pl.ds(c * chunk, chunk)]],
                             rows_v.at[pl.ds(c * chunk, chunk)], gsems[c])
            for c in range(n_chunks)
        ]
        stores = []
        for c in range(n_chunks):
            gathers[c].wait()
            stores.append(
                pltpu.async_copy(rows_v.at[pl.ds(c * chunk, chunk)],
                                 out_hbm.at[pl.ds(base + c * chunk, chunk)],
                                 ssems[c]))
        for s in stores:
            s.wait()

    return gather_kernel


@jax.jit
def kernel(x, t, embedding):
    del x
    idx = jnp.squeeze(t, axis=-1).astype(jnp.int32)
    table4 = embedding.astype(jnp.float32).reshape(
        _TIME_STEPS, _DIM // 128, 1, 128)
    out = _build_gather()(idx, table4)
    return out.reshape(_BATCH, _DIM, 1, 1)


# speedup vs baseline: 1.0388x; 1.0388x over previous
import functools

import jax
import jax.numpy as jnp
from jax import lax
from jax.experimental import pallas as pl
from jax.experimental.pallas import tpu as pltpu
from jax.experimental.pallas import tpu_sc as plsc

_TIME_STEPS = 1000
_DIM = 512
_BATCH = 4096


@functools.cache
def _build_gather():
    info = plsc.get_sparse_core_info()
    num_cores, num_subcores = info.num_cores, info.num_subcores
    num_workers = num_cores * num_subcores
    rows_per_worker = _BATCH // num_workers
    assert _BATCH % (8 * num_workers) == 0

    mesh = plsc.VectorSubcoreMesh(core_axis_name="c", subcore_axis_name="s")

    chunk = 64
    n_chunks = rows_per_worker // chunk

    @functools.partial(
        pl.kernel,
        mesh=mesh,
        out_type=jax.ShapeDtypeStruct((_BATCH, _DIM // 128, 1, 128),
                                      jnp.float32),
        scratch_types=[
            pltpu.VMEM((rows_per_worker,), jnp.int32),
            pltpu.VMEM((rows_per_worker, _DIM // 128, 1, 128), jnp.float32),
            pltpu.SemaphoreType.DMA,
            [pltpu.SemaphoreType.DMA] * n_chunks,
            [pltpu.SemaphoreType.DMA] * n_chunks,
        ],
    )
    def gather_kernel(idx_hbm, table_hbm, out_hbm, idx_v, rows_v, isem,
                      gsems, ssems):
        wid = lax.axis_index("s") * num_cores + lax.axis_index("c")
        base = wid * rows_per_worker
        pltpu.async_copy(idx_hbm.at[pl.ds(base, rows_per_worker)], idx_v,
                         isem).wait()
        gathers = [
            pltpu.async_copy(table_hbm.at[idx_v.at[pl.ds(c * chunk, chunk)]],
                             rows_v.at[pl.ds(c * chunk, chunk)], gsems[c])
            for c in range(n_chunks)
        ]
        stores = []
        for c in range(n_chunks):
            gathers[c].wait()
            stores.append(
                pltpu.async_copy(rows_v.at[pl.ds(c * chunk, chunk)],
                                 out_hbm.at[pl.ds(base + c * chunk, chunk)],
                                 ssems[c]))
        for s in stores:
            s.wait()

    return gather_kernel


@jax.jit
def kernel(x, t, embedding):
    del x
    idx = jnp.squeeze(t, axis=-1).astype(jnp.int32)
    table4 = embedding.astype(jnp.float32).reshape(
        _TIME_STEPS, _DIM // 128, 1, 128)
    out = _build_gather()(idx, table4)
    return out.reshape(_BATCH, _DIM, 1, 1)


# speedup vs baseline: 1.0514x; 1.0122x over previous
import functools

import jax
import jax.numpy as jnp
from jax import lax
from jax.experimental import pallas as pl
from jax.experimental.pallas import tpu as pltpu
from jax.experimental.pallas import tpu_sc as plsc

_TIME_STEPS = 1000
_DIM = 512
_BATCH = 4096
_SUB = _DIM // 128


@functools.cache
def _build_gather():
    info = plsc.get_sparse_core_info()
    num_cores, num_subcores = info.num_cores, info.num_subcores
    num_workers = num_cores * num_subcores
    rows_per_worker = _BATCH // num_workers
    sub_per_worker = rows_per_worker * _SUB
    assert _BATCH % (8 * num_workers) == 0
    n_chunks = sub_per_worker // 128

    mesh = plsc.VectorSubcoreMesh(core_axis_name="c", subcore_axis_name="s")

    @functools.partial(
        pl.kernel,
        mesh=mesh,
        out_type=jax.ShapeDtypeStruct((_BATCH * _SUB, 128), jnp.float32),
        scratch_types=[
            pltpu.VMEM((sub_per_worker,), jnp.int32),
            pltpu.VMEM((sub_per_worker, 128), jnp.float32),
            pltpu.SemaphoreType.DMA,
            [pltpu.SemaphoreType.DMA] * n_chunks,
        ],
    )
    def gather_kernel(idx_hbm, table_hbm, out_hbm, idx_v, rows_v, sem, gsems):
        wid = lax.axis_index("s") * num_cores + lax.axis_index("c")
        base = wid * sub_per_worker
        pltpu.async_copy(idx_hbm.at[pl.ds(base, sub_per_worker)], idx_v,
                         sem).wait()
        gathers = [
            pltpu.async_copy(table_hbm.at[idx_v.at[pl.ds(c * 128, 128)]],
                             rows_v.at[pl.ds(c * 128, 128)], gsems[c])
            for c in range(n_chunks)
        ]
        for g in gathers:
            g.wait()
        pltpu.sync_copy(rows_v, out_hbm.at[pl.ds(base, sub_per_worker)])

    return gather_kernel


@jax.jit
def kernel(x, t, embedding):
    del x
    idx = jnp.squeeze(t, axis=-1).astype(jnp.int32)
    idx4 = (idx[:, None] * _SUB + jnp.arange(_SUB, dtype=jnp.int32)).reshape(-1)
    table2 = embedding.astype(jnp.float32).reshape(_TIME_STEPS * _SUB, 128)
    out = _build_gather()(idx4, table2)
    return out.reshape(_BATCH, _DIM, 1, 1)
